# baseline (speedup 1.0000x reference)
import functools

import jax
import jax.numpy as jnp
from jax import lax
from jax.experimental import pallas as pl
from jax.experimental.pallas import tpu as pltpu
from jax.experimental.pallas import tpu_sc as plsc


@functools.lru_cache(maxsize=None)
def _make_flip_gather(L: int, H: int):
    info = plsc.get_sparse_core_info()
    NC, NS, LANES = info.num_cores, info.num_subcores, info.num_lanes
    NW = NC * NS
    rows_per_w = L // NW
    C = 32
    n_chunks = rows_per_w // C

    mesh = plsc.VectorSubcoreMesh(core_axis_name="c", subcore_axis_name="s")

    @functools.partial(
        pl.kernel,
        mesh=mesh,
        out_type=jax.ShapeDtypeStruct((L, H), jnp.float32),
        scratch_types=[
            pltpu.VMEM((n_chunks, C), jnp.int32),
            pltpu.VMEM((2, C, H), jnp.float32),
            pltpu.SemaphoreType.DMA,
            pltpu.SemaphoreType.DMA,
            pltpu.SemaphoreType.DMA,
            pltpu.SemaphoreType.DMA,
        ],
    )
    def flip_gather(table_hbm, out_hbm, idx_v, buf_v, gs0, gs1, ws0, ws1):
        gsem = (gs0, gs1)
        wsem = (ws0, ws1)
        wid = lax.axis_index("s") * NC + lax.axis_index("c")
        base = wid * rows_per_w
        top = (L - 1) - base
        for c in range(n_chunks):
            for i in range(C // LANES):
                idx_v[c, pl.ds(i * LANES, LANES)] = (
                    (top - c * C - i * LANES) - lax.iota(jnp.int32, LANES)
                )
        gops, wops = {}, {}
        gops[0] = pltpu.async_copy(table_hbm.at[idx_v.at[0]], buf_v.at[0], gsem[0])
        for c in range(n_chunks):
            b = c % 2
            gops[c].wait()
            if c + 1 < n_chunks:
                if c >= 1:
                    wops[c - 1].wait()
                gops[c + 1] = pltpu.async_copy(
                    table_hbm.at[idx_v.at[c + 1]], buf_v.at[1 - b], gsem[1 - b]
                )
            wops[c] = pltpu.async_copy(
                buf_v.at[b], out_hbm.at[pl.ds(base + c * C, C)], wsem[b]
            )
        wops[n_chunks - 2].wait()
        wops[n_chunks - 1].wait()

    return flip_gather


def kernel(hidden_states, pos_table):
    L = hidden_states.shape[1]
    H = pos_table.shape[1]
    out = _make_flip_gather(L, H)(pos_table)
    return out.reshape(1, L, H)


# speedup vs baseline: 1.0387x; 1.0387x over previous
import functools

import jax
import jax.numpy as jnp
from jax import lax
from jax.experimental import pallas as pl
from jax.experimental.pallas import tpu as pltpu
from jax.experimental.pallas import tpu_sc as plsc


@functools.lru_cache(maxsize=None)
def _make_flip_gather(L: int, H: int):
    info = plsc.get_sparse_core_info()
    NC, NS, LANES = info.num_cores, info.num_subcores, info.num_lanes
    NW = NC * NS
    rows_per_w = L // NW
    C = 32
    NBUF = 3
    n_chunks = rows_per_w // C

    mesh = plsc.VectorSubcoreMesh(core_axis_name="c", subcore_axis_name="s")

    @functools.partial(
        pl.kernel,
        mesh=mesh,
        out_type=jax.ShapeDtypeStruct((L, H), jnp.float32),
        scratch_types=[
            pltpu.VMEM((n_chunks, C), jnp.int32),
            pltpu.VMEM((NBUF, C, H), jnp.float32),
            pltpu.SemaphoreType.DMA,
            pltpu.SemaphoreType.DMA,
            pltpu.SemaphoreType.DMA,
            pltpu.SemaphoreType.DMA,
            pltpu.SemaphoreType.DMA,
            pltpu.SemaphoreType.DMA,
        ],
    )
    def flip_gather(table_hbm, out_hbm, idx_v, buf_v, g0, g1, g2, w0, w1, w2):
        gsem = (g0, g1, g2)
        wsem = (w0, w1, w2)
        wid = lax.axis_index("s") * NC + lax.axis_index("c")
        base = wid * rows_per_w
        top = (L - 1) - base
        for c in range(n_chunks):
            for i in range(C // LANES):
                idx_v[c, pl.ds(i * LANES, LANES)] = (
                    (top - c * C - i * LANES) - lax.iota(jnp.int32, LANES)
                )

        def gather(c):
            b = c % NBUF
            return pltpu.async_copy(
                table_hbm.at[idx_v.at[c]], buf_v.at[b], gsem[b]
            )

        gops, wops = {}, {}
        for c in range(min(2, n_chunks)):
            gops[c] = gather(c)
        for c in range(n_chunks):
            b = c % NBUF
            gops[c].wait()
            wops[c] = pltpu.async_copy(
                buf_v.at[b], out_hbm.at[pl.ds(base + c * C, C)], wsem[b]
            )
            if c + 2 < n_chunks:
                if c >= 1:
                    wops[c - 1].wait()
                gops[c + 2] = gather(c + 2)
        wops[n_chunks - 3].wait()
        wops[n_chunks - 2].wait()
        wops[n_chunks - 1].wait()

    return flip_gather


def kernel(hidden_states, pos_table):
    L = hidden_states.shape[1]
    H = pos_table.shape[1]
    out = _make_flip_gather(L, H)(pos_table)
    return out.reshape(1, L, H)
